# baseline (speedup 1.0000x reference)
import functools
import jax
import jax.numpy as jnp
from jax import lax
from jax.experimental import pallas as pl
from jax.experimental.pallas import tpu as pltpu
from jax.experimental.pallas import tpu_sc as plsc

B = 4
N = 10000
T = 12
D = 128
ROWS = B * N
NW = 32
RPW = ROWS // NW
CH = 25
NCH = RPW // CH


def _sc_kernel(x_hbm, emb_hbm, o_hbm, xbuf, ebuf):
    nc = lax.axis_size("c")
    wid = lax.axis_index("s") * nc + lax.axis_index("c")
    base0 = wid * RPW

    def chunk_body(c, _):
        base = base0 + c * CH
        v0 = lax.rem(base, N)
        v0a = (v0 // 8) * 8
        off = v0 - v0a
        pltpu.sync_copy(x_hbm.at[pl.ds(base, CH)], xbuf)
        pltpu.sync_copy(emb_hbm.at[pl.ds(v0a, 32)], ebuf)

        def row_body(r, _):
            for kk in range(D // 16):
                sl = pl.ds(kk * 16, 16)
                ev = ebuf[r + off, sl]
                for t in range(T):
                    xbuf[r, t, sl] = xbuf[r, t, sl] + ev
            return 0

        lax.fori_loop(0, CH, row_body, 0)
        pltpu.sync_copy(xbuf, o_hbm.at[pl.ds(base, CH)])
        return 0

    lax.fori_loop(0, NCH, chunk_body, 0)


def kernel(x, emb_weight):
    batch, n, t, d = x.shape
    x2 = x.reshape(batch * n, t, d)
    mesh = plsc.VectorSubcoreMesh(core_axis_name="c", subcore_axis_name="s")
    run = pl.kernel(
        _sc_kernel,
        mesh=mesh,
        out_type=jax.ShapeDtypeStruct((batch * n, t, d), x.dtype),
        scratch_types=[
            pltpu.VMEM((CH, t, d), x.dtype),
            pltpu.VMEM((32, d), x.dtype),
        ],
    )
    out = run(x2, emb_weight)
    return out.reshape(batch, n, t, d)


# speedup vs baseline: 1.2291x; 1.2291x over previous
import jax
import jax.numpy as jnp
from jax import lax
from jax.experimental import pallas as pl
from jax.experimental.pallas import tpu as pltpu
from jax.experimental.pallas import tpu_sc as plsc

B = 4
N = 10000
T = 12
D = 128
ROWS = B * N
NW = 32
RPW = ROWS // NW
CH = 10
NCH = RPW // CH
NBUF = 4
EB = 24


def _sc_kernel(x_hbm, emb_hbm, o_hbm, xbuf, ebuf, sx, se, so):
    nc = lax.axis_size("c")
    wid = lax.axis_index("s") * nc + lax.axis_index("c")
    base0 = wid * RPW

    def xcopy(c, slot):
        base = base0 + c * CH
        return pltpu.make_async_copy(
            x_hbm.at[pl.ds(base, CH)], xbuf.at[slot], sx.at[slot]
        )

    def ecopy(c, slot):
        base = base0 + c * CH
        v0a = (lax.rem(base, N) // 8) * 8
        return pltpu.make_async_copy(
            emb_hbm.at[pl.ds(v0a, EB)], ebuf.at[slot], se.at[slot]
        )

    def ocopy(c, slot):
        base = base0 + c * CH
        return pltpu.make_async_copy(
            xbuf.at[slot], o_hbm.at[pl.ds(base, CH)], so.at[slot]
        )

    for c in range(3):
        xcopy(c, c).start()
        ecopy(c, c).start()

    def body(c, _):
        slot = lax.rem(c, NBUF)
        xcopy(c, slot).wait()
        ecopy(c, slot).wait()
        base = base0 + c * CH
        v0 = lax.rem(base, N)
        off = v0 - (v0 // 8) * 8

        def row(r, _):
            for kk in range(D // 16):
                sl = pl.ds(kk * 16, 16)
                ev = ebuf[slot, r + off, sl]
                for t in range(T):
                    plsc.addupdate(xbuf.at[slot, r, t, sl], ev)
            return 0

        lax.fori_loop(0, CH, row, 0)
        ocopy(c, slot).start()

        @pl.when(c + 3 < NCH)
        def _():
            nslot = lax.rem(c + 3, NBUF)

            @pl.when(c >= 1)
            def _():
                ocopy(c - 1, nslot).wait()

            xcopy(c + 3, nslot).start()
            ecopy(c + 3, nslot).start()

        return 0

    lax.fori_loop(0, NCH, body, 0)
    for c in range(NCH - 4, NCH):
        ocopy(c, c % NBUF).wait()


def kernel(x, emb_weight):
    batch, n, t, d = x.shape
    x2 = x.reshape(batch * n, t, d)
    mesh = plsc.VectorSubcoreMesh(core_axis_name="c", subcore_axis_name="s")
    run = pl.kernel(
        _sc_kernel,
        mesh=mesh,
        out_type=jax.ShapeDtypeStruct((batch * n, t, d), x.dtype),
        scratch_types=[
            pltpu.VMEM((NBUF, CH, t, d), x.dtype),
            pltpu.VMEM((NBUF, EB, d), x.dtype),
            pltpu.SemaphoreType.DMA((NBUF,)),
            pltpu.SemaphoreType.DMA((NBUF,)),
            pltpu.SemaphoreType.DMA((NBUF,)),
        ],
    )
    out = run(x2, emb_weight)
    return out.reshape(batch, n, t, d)


# speedup vs baseline: 1.4998x; 1.2203x over previous
import jax
import jax.numpy as jnp
from jax import lax
from jax.experimental import pallas as pl
from jax.experimental.pallas import tpu as pltpu
from jax.experimental.pallas import tpu_sc as plsc

B = 4
N = 10000
T = 12
D = 128
ROWS = B * N
NW = 32
RPW = ROWS // NW
CH = 10
NCH = RPW // CH
NBUF = 4
EB = 24


def _sc_kernel(x_hbm, emb_hbm, o_hbm, xbuf, ebuf, sx, se, so):
    nc = lax.axis_size("c")
    wid = lax.axis_index("s") * nc + lax.axis_index("c")
    base0 = wid * RPW
    bb = base0 // N
    n00 = lax.rem(base0, N)

    def xcopy(c, slot):
        n0 = n00 + c * CH
        return pltpu.make_async_copy(
            x_hbm.at[bb, pl.ds(n0, CH)], xbuf.at[slot], sx.at[slot]
        )

    def ecopy(c, slot):
        v0a = ((n00 + c * CH) // 8) * 8
        return pltpu.make_async_copy(
            emb_hbm.at[pl.ds(v0a, EB)], ebuf.at[slot], se.at[slot]
        )

    def ocopy(c, slot):
        n0 = n00 + c * CH
        return pltpu.make_async_copy(
            xbuf.at[slot], o_hbm.at[bb, pl.ds(n0, CH)], so.at[slot]
        )

    for c in range(3):
        xcopy(c, c).start()
        ecopy(c, c).start()

    def body(c, _):
        slot = lax.rem(c, NBUF)
        xcopy(c, slot).wait()
        ecopy(c, slot).wait()
        v0 = n00 + c * CH
        off = v0 - (v0 // 8) * 8

        def row(r, _):
            for kk in range(D // 16):
                sl = pl.ds(kk * 16, 16)
                ev = ebuf[slot, r + off, sl]
                for t in range(T):
                    plsc.addupdate(xbuf.at[slot, r, t, sl], ev)
            return 0

        lax.fori_loop(0, CH, row, 0)
        ocopy(c, slot).start()

        @pl.when(c + 3 < NCH)
        def _():
            nslot = lax.rem(c + 3, NBUF)

            @pl.when(c >= 1)
            def _():
                ocopy(c - 1, nslot).wait()

            xcopy(c + 3, nslot).start()
            ecopy(c + 3, nslot).start()

        return 0

    lax.fori_loop(0, NCH, body, 0)
    for c in range(NCH - 4, NCH):
        ocopy(c, c % NBUF).wait()


def kernel(x, emb_weight):
    batch, n, t, d = x.shape
    mesh = plsc.VectorSubcoreMesh(core_axis_name="c", subcore_axis_name="s")
    run = pl.kernel(
        _sc_kernel,
        mesh=mesh,
        out_type=jax.ShapeDtypeStruct((batch, n, t, d), x.dtype),
        scratch_types=[
            pltpu.VMEM((NBUF, CH, t, d), x.dtype),
            pltpu.VMEM((NBUF, EB, d), x.dtype),
            pltpu.SemaphoreType.DMA((NBUF,)),
            pltpu.SemaphoreType.DMA((NBUF,)),
            pltpu.SemaphoreType.DMA((NBUF,)),
        ],
    )
    return run(x, emb_weight)


# speedup vs baseline: 3.4570x; 2.3049x over previous
import jax
import jax.numpy as jnp
from jax.experimental import pallas as pl
from jax.experimental.pallas import tpu as pltpu

BN = 2000


def _add_kernel(x_ref, emb_ref, o_ref):
    o_ref[...] = x_ref[...] + emb_ref[...][None, :, :]


def kernel(x, emb_weight):
    batch, n, t, d = x.shape
    xt = jnp.transpose(x, (0, 2, 1, 3)).reshape(batch * t, n, d)
    out = pl.pallas_call(
        _add_kernel,
        grid=(n // BN, batch * t),
        in_specs=[
            pl.BlockSpec((1, BN, d), lambda j, s: (s, j, 0)),
            pl.BlockSpec((BN, d), lambda j, s: (j, 0)),
        ],
        out_specs=pl.BlockSpec((1, BN, d), lambda j, s: (s, j, 0)),
        out_shape=jax.ShapeDtypeStruct((batch * t, n, d), x.dtype),
        compiler_params=pltpu.CompilerParams(
            dimension_semantics=("parallel", "parallel"),
        ),
    )(xt, emb_weight)
    return jnp.transpose(out.reshape(batch, t, n, d), (0, 2, 1, 3))


# speedup vs baseline: 5.1087x; 1.4778x over previous
import jax
import jax.numpy as jnp
from jax.experimental import pallas as pl
from jax.experimental.pallas import tpu as pltpu

BN = 5000


def _add_kernel(x_ref, emb_ref, o_ref):
    o_ref[...] = x_ref[...] + emb_ref[...][None, :, :]


def kernel(x, emb_weight):
    batch, n, t, d = x.shape
    xt = jnp.transpose(x, (0, 2, 1, 3)).reshape(batch * t, n, d)
    out = pl.pallas_call(
        _add_kernel,
        grid=(n // BN, batch * t),
        in_specs=[
            pl.BlockSpec((1, BN, d), lambda j, s: (s, j, 0)),
            pl.BlockSpec((BN, d), lambda j, s: (j, 0)),
        ],
        out_specs=pl.BlockSpec((1, BN, d), lambda j, s: (s, j, 0)),
        out_shape=jax.ShapeDtypeStruct((batch * t, n, d), x.dtype),
        compiler_params=pltpu.CompilerParams(
            dimension_semantics=("parallel", "parallel"),
        ),
    )(xt, emb_weight)
    return jnp.transpose(out.reshape(batch, t, n, d), (0, 2, 1, 3))
